# Initial kernel scaffold; baseline (speedup 1.0000x reference)
import jax
import jax.numpy as jnp
from jax.experimental import pallas as pl


def kernel(nodes, edge_index, edge_features, W1, b1, W2, b2, W3, b3):
    raise NotImplementedError("write your pallas kernel here")


# speedup vs baseline: 2.1303x; 2.1303x over previous
import functools

import jax
import jax.numpy as jnp
from jax import lax
from jax.experimental import pallas as pl
from jax.experimental.pallas import tpu as pltpu
from jax.experimental.pallas import tpu_sc as plsc

N, E, D = 10000, 320000, 128
L = 16
NC, NS = 2, 16
NW = NC * NS
EW = E // NW
C = 80
CHUNKS = EW // C
NP = 10240
ROWS_PER_TILE = NP // NS

_HI = lax.Precision.HIGHEST


def _dot(a, b):
    return lax.dot_general(a, b, (((1,), (0,)), ((), ())),
                           precision=_HI, preferred_element_type=jnp.float32)


def _silu(x):
    return x * jax.nn.sigmoid(x)


def _proj_body(nodes_ref, w_ref, b_ref, out_ref):
    out_ref[...] = _dot(nodes_ref[...], w_ref[...]) + b_ref[...]


def _node_proj(nodes, w, b):
    return pl.pallas_call(
        _proj_body,
        out_shape=jax.ShapeDtypeStruct((N, D), jnp.float32),
    )(nodes, w, b.reshape(1, D))


_BE = 1600


def _edge_proj_body(ef_ref, w_ref, out_ref):
    out_ref[...] = _dot(ef_ref[...], w_ref[...])


def _edge_proj(ef, w):
    grid = E // _BE
    return pl.pallas_call(
        _edge_proj_body,
        grid=(grid,),
        in_specs=[
            pl.BlockSpec((_BE, D), lambda i: (i, 0)),
            pl.BlockSpec((D, D), lambda i: (0, 0)),
        ],
        out_specs=pl.BlockSpec((_BE, D), lambda i: (i, 0)),
        out_shape=jax.ShapeDtypeStruct((E, D), jnp.float32),
    )(ef, w)


def _sc_body(p_hbm, efp_hbm, src_hbm, tgt_hbm, zeros_hbm, out_hbm,
             sidx, tidx, grows, erows, acc, sem):
    c = lax.axis_index("c")
    s = lax.axis_index("s")
    wid = s * NC + c
    row0 = s * ROWS_PER_TILE

    pltpu.sync_copy(zeros_hbm.at[pl.ds(row0, ROWS_PER_TILE)],
                    acc.at[pl.ds(row0, ROWS_PER_TILE)])
    plsc.subcore_barrier()

    base = wid * EW

    def chunk_body(t, carry):
        eb = base + t * C
        pltpu.sync_copy(src_hbm.at[pl.ds(eb, C)], sidx)
        pltpu.sync_copy(tgt_hbm.at[pl.ds(eb, C)], tidx)
        pltpu.async_copy(p_hbm.at[sidx], grows, sem).wait()
        pltpu.sync_copy(efp_hbm.at[pl.ds(eb, C)], erows)

        def row_body(i, carry2):
            for j in range(D // L):
                x = grows[i, pl.ds(j * L, L)] + erows[i, pl.ds(j * L, L)]
                erows[i, pl.ds(j * L, L)] = x / (1.0 + jnp.exp(-x))
            return carry2

        lax.fori_loop(0, C, row_body, 0)
        pltpu.sync_copy(erows, acc.at[tidx], add=True)
        return carry

    lax.fori_loop(0, CHUNKS, chunk_body, 0)
    plsc.subcore_barrier()
    pltpu.sync_copy(acc.at[pl.ds(row0, ROWS_PER_TILE)],
                    out_hbm.at[c, pl.ds(row0, ROWS_PER_TILE)])


@functools.partial(
    pl.kernel,
    out_type=jax.ShapeDtypeStruct((NC, NP, D), jnp.float32),
    mesh=plsc.VectorSubcoreMesh(core_axis_name="c", subcore_axis_name="s"),
    scratch_types=[
        pltpu.VMEM((C,), jnp.int32),
        pltpu.VMEM((C,), jnp.int32),
        pltpu.VMEM((C, D), jnp.float32),
        pltpu.VMEM((C, D), jnp.float32),
        pltpu.VMEM_SHARED((NP, D), jnp.float32),
        pltpu.SemaphoreType.DMA,
    ],
)
def _sc_aggregate(p_hbm, efp_hbm, src_hbm, tgt_hbm, zeros_hbm, out_hbm,
                  sidx, tidx, grows, erows, acc, sem):
    _sc_body(p_hbm, efp_hbm, src_hbm, tgt_hbm, zeros_hbm, out_hbm,
             sidx, tidx, grows, erows, acc, sem)


_BN = 2000


def _update_body(nodes_ref, a0_ref, a1_ref, w2a_ref, w2b_ref, b2_ref,
                 w3_ref, b3_ref, out_ref):
    nodes = nodes_ref[...]
    agg = a0_ref[...] + a1_ref[...]
    u = _silu(_dot(nodes, w2a_ref[...]) + _dot(agg, w2b_ref[...]) + b2_ref[...])
    out_ref[...] = nodes + _dot(u, w3_ref[...]) + b3_ref[...]


def _node_update(nodes, a0, a1, w2a, w2b, b2, w3, b3):
    grid = N // _BN
    blk = lambda i: (i, 0)
    whole = lambda i: (0, 0)
    return pl.pallas_call(
        _update_body,
        grid=(grid,),
        in_specs=[
            pl.BlockSpec((_BN, D), blk),
            pl.BlockSpec((_BN, D), blk),
            pl.BlockSpec((_BN, D), blk),
            pl.BlockSpec((D, D), whole),
            pl.BlockSpec((D, D), whole),
            pl.BlockSpec((1, D), whole),
            pl.BlockSpec((D, D), whole),
            pl.BlockSpec((1, D), whole),
        ],
        out_specs=pl.BlockSpec((_BN, D), blk),
        out_shape=jax.ShapeDtypeStruct((N, D), jnp.float32),
    )(nodes, a0, a1, w2a, w2b, b2.reshape(1, D), w3, b3.reshape(1, D))


def kernel(nodes, edge_index, edge_features, W1, b1, W2, b2, W3, b3):
    src = edge_index[0]
    tgt = edge_index[1]
    W1a, W1b = W1[:D], W1[D:]
    W2a, W2b = W2[:D], W2[D:]

    p = _node_proj(nodes, W1a, b1)
    efp = _edge_proj(edge_features, W1b)
    zeros = jnp.zeros((NP, D), jnp.float32)
    partials = _sc_aggregate(p, efp, src, tgt, zeros)
    return _node_update(nodes, partials[0, :N], partials[1, :N],
                        W2a, W2b, b2, W3, b3)


# speedup vs baseline: 3.6318x; 1.7049x over previous
import functools

import jax
import jax.numpy as jnp
from jax import lax
from jax.experimental import pallas as pl
from jax.experimental.pallas import tpu as pltpu
from jax.experimental.pallas import tpu_sc as plsc

N, E, D = 10000, 320000, 128
L = 16
NC, NS = 2, 16
NW = NC * NS
EW = E // NW
C = 40
CHUNKS = EW // C
PAIRS = CHUNKS // 2

_HI = lax.Precision.HIGHEST


def _dot(a, b):
    return lax.dot_general(a, b, (((1,), (0,)), ((), ())),
                           precision=_HI, preferred_element_type=jnp.float32)


def _silu(x):
    return x * jax.nn.sigmoid(x)


def _proj_body(nodes_ref, w_ref, b_ref, out_ref):
    out_ref[...] = _dot(nodes_ref[...], w_ref[...]) + b_ref[...]


def _node_proj(nodes, w, b):
    return pl.pallas_call(
        _proj_body,
        out_shape=jax.ShapeDtypeStruct((N, D), jnp.float32),
    )(nodes, w, b.reshape(1, D))


_BE = 1600


def _edge_proj_body(ef_ref, w_ref, out_ref):
    out_ref[...] = _dot(ef_ref[...], w_ref[...])


def _edge_proj(ef, w):
    grid = E // _BE
    return pl.pallas_call(
        _edge_proj_body,
        grid=(grid,),
        in_specs=[
            pl.BlockSpec((_BE, D), lambda i: (i, 0)),
            pl.BlockSpec((D, D), lambda i: (0, 0)),
        ],
        out_specs=pl.BlockSpec((_BE, D), lambda i: (i, 0)),
        out_shape=jax.ShapeDtypeStruct((E, D), jnp.float32),
    )(ef, w)


_ZR = 640
_ZR_LAST = N - 15 * _ZR


def _sc_body(p_hbm, efp_hbm, src_hbm, tgt_hbm, zeros_hbm, out_hbm,
             srcall, tidx0, tidx1, g0, g1, e0, e1,
             acc, tsems, gsems, esems, ssems):
    c = lax.axis_index("c")
    s = lax.axis_index("s")
    wid = s * NC + c
    base = wid * EW
    tidx = (tidx0, tidx1)
    grows = (g0, g1)
    erows = (e0, e1)

    @pl.when(s < NS - 1)
    def _():
        pltpu.sync_copy(zeros_hbm.at[pl.ds(s * _ZR, _ZR)],
                        acc.at[pl.ds(s * _ZR, _ZR)])

    @pl.when(s == NS - 1)
    def _():
        pltpu.sync_copy(zeros_hbm.at[pl.ds(15 * _ZR, _ZR_LAST)],
                        acc.at[pl.ds(15 * _ZR, _ZR_LAST)])

    pltpu.sync_copy(src_hbm.at[pl.ds(base, EW)], srcall)
    plsc.subcore_barrier()

    def fetch(b, t):
        eb = base + t * C
        pltpu.async_copy(tgt_hbm.at[pl.ds(eb, C)], tidx[b], tsems.at[b])
        pltpu.async_copy(p_hbm.at[srcall.at[pl.ds(t * C, C)]], grows[b],
                         gsems.at[b])
        pltpu.async_copy(efp_hbm.at[pl.ds(eb, C)], erows[b], esems.at[b])

    def wait_fetch(b, t):
        eb = base + t * C
        pltpu.make_async_copy(tgt_hbm.at[pl.ds(eb, C)], tidx[b],
                              tsems.at[b]).wait()
        pltpu.make_async_copy(p_hbm.at[srcall.at[pl.ds(t * C, C)]], grows[b],
                              gsems.at[b]).wait()
        pltpu.make_async_copy(efp_hbm.at[pl.ds(eb, C)], erows[b],
                              esems.at[b]).wait()

    def compute(b):
        gb = grows[b]
        eb_ = erows[b]

        def row_body(i):
            for j in range(D // L):
                x = gb[i, pl.ds(j * L, L)] + eb_[i, pl.ds(j * L, L)]
                eb_[i, pl.ds(j * L, L)] = x / (1.0 + jnp.exp(-x))

        plsc.parallel_loop(0, C, unroll=2)(row_body)

    def process(b, t):
        wait_fetch(b, t)
        compute(b)
        pltpu.async_copy(erows[b], acc.at[tidx[b]], ssems.at[b], add=True)
        pltpu.make_async_copy(erows[b], acc.at[tidx[b]], ssems.at[b]).wait()

        @pl.when(t + 2 < CHUNKS)
        def _():
            fetch(b, t + 2)

    fetch(0, 0)
    fetch(1, 1)

    def pair_body(it, carry):
        process(0, it * 2)
        process(1, it * 2 + 1)
        return carry

    lax.fori_loop(0, PAIRS, pair_body, 0)

    plsc.subcore_barrier()

    @pl.when(s < NS - 1)
    def _():
        pltpu.sync_copy(acc.at[pl.ds(s * _ZR, _ZR)],
                        out_hbm.at[c, pl.ds(s * _ZR, _ZR)])

    @pl.when(s == NS - 1)
    def _():
        pltpu.sync_copy(acc.at[pl.ds(15 * _ZR, _ZR_LAST)],
                        out_hbm.at[c, pl.ds(15 * _ZR, _ZR_LAST)])


@functools.partial(
    pl.kernel,
    out_type=jax.ShapeDtypeStruct((NC, N, D), jnp.float32),
    mesh=plsc.VectorSubcoreMesh(core_axis_name="c", subcore_axis_name="s"),
    scratch_types=[
        pltpu.VMEM((EW,), jnp.int32),
        pltpu.VMEM((C,), jnp.int32),
        pltpu.VMEM((C,), jnp.int32),
        pltpu.VMEM((C, D), jnp.float32),
        pltpu.VMEM((C, D), jnp.float32),
        pltpu.VMEM((C, D), jnp.float32),
        pltpu.VMEM((C, D), jnp.float32),
        pltpu.VMEM_SHARED((N, D), jnp.float32),
        pltpu.SemaphoreType.DMA((2,)),
        pltpu.SemaphoreType.DMA((2,)),
        pltpu.SemaphoreType.DMA((2,)),
        pltpu.SemaphoreType.DMA((2,)),
    ],
)
def _sc_aggregate(*args):
    _sc_body(*args)


_BN = 2000


def _update_body(nodes_ref, part_ref, w2a_ref, w2b_ref, b2_ref,
                 w3_ref, b3_ref, out_ref):
    nodes = nodes_ref[...]
    agg = part_ref[0] + part_ref[1]
    u = _silu(_dot(nodes, w2a_ref[...]) + _dot(agg, w2b_ref[...]) + b2_ref[...])
    out_ref[...] = nodes + _dot(u, w3_ref[...]) + b3_ref[...]


def _node_update(nodes, partials, w2a, w2b, b2, w3, b3):
    grid = N // _BN
    blk = lambda i: (i, 0)
    whole = lambda i: (0, 0)
    return pl.pallas_call(
        _update_body,
        grid=(grid,),
        in_specs=[
            pl.BlockSpec((_BN, D), blk),
            pl.BlockSpec((NC, _BN, D), lambda i: (0, i, 0)),
            pl.BlockSpec((D, D), whole),
            pl.BlockSpec((D, D), whole),
            pl.BlockSpec((1, D), whole),
            pl.BlockSpec((D, D), whole),
            pl.BlockSpec((1, D), whole),
        ],
        out_specs=pl.BlockSpec((_BN, D), blk),
        out_shape=jax.ShapeDtypeStruct((N, D), jnp.float32),
    )(nodes, partials, w2a, w2b, b2.reshape(1, D), w3, b3.reshape(1, D))


def kernel(nodes, edge_index, edge_features, W1, b1, W2, b2, W3, b3):
    src = edge_index[0]
    tgt = edge_index[1]
    W1a, W1b = W1[:D], W1[D:]
    W2a, W2b = W2[:D], W2[D:]

    p = _node_proj(nodes, W1a, b1)
    efp = _edge_proj(edge_features, W1b)
    zeros = jnp.zeros((N, D), jnp.float32)
    partials = _sc_aggregate(p, efp, src, tgt, zeros)
    return _node_update(nodes, partials, W2a, W2b, b2, W3, b3)
